# Initial kernel scaffold; baseline (speedup 1.0000x reference)
import jax
import jax.numpy as jnp
from jax.experimental import pallas as pl


def kernel(xyz_hi, xyz_lo, feat_skip, feat_lo, W1, b1, W2, b2, gamma, beta):
    raise NotImplementedError("write your pallas kernel here")


# speedup vs baseline: 26.0174x; 26.0174x over previous
import functools

import jax
import jax.numpy as jnp
from jax import lax
from jax.experimental import pallas as pl
from jax.experimental.pallas import tpu as pltpu
from jax.experimental.pallas import tpu_sc as plsc


def _knn_body(hi_ref, lo_ref, idx_ref, w_ref, *, S):
    b = pl.program_id(0)
    hi = hi_ref[0]
    lo = lo_ref[0]
    bn = hi.shape[0]

    a2 = (hi[:, 0:1] * hi[:, 0:1] + hi[:, 1:2] * hi[:, 1:2]
          + hi[:, 2:3] * hi[:, 2:3])
    b2 = (lo[0:1, :] * lo[0:1, :] + lo[1:2, :] * lo[1:2, :]
          + lo[2:3, :] * lo[2:3, :])
    ab = jnp.dot(hi.astype(jnp.bfloat16), lo.astype(jnp.bfloat16),
                 preferred_element_type=jnp.float32)
    d2 = jnp.maximum(a2 + b2 - 2.0 * ab, 0.0)

    iota = lax.broadcasted_iota(jnp.int32, (bn, S), 1)
    BIG = jnp.float32(3.0e38)
    dists = []
    for k in range(3):
        m = jnp.min(d2, axis=1, keepdims=True)
        ismin = d2 <= m
        first = jnp.min(jnp.where(ismin, iota, S), axis=1, keepdims=True)
        sel = iota == first
        d2 = jnp.where(sel, BIG, d2)
        idx_ref[0, 0, k, :] = first[:, 0] + b * S
        dists.append(jnp.sqrt(jnp.maximum(m[:, 0], 0.0)))
    inv = [1.0 / (d + 1e-8) for d in dists]
    wsum = inv[0] + inv[1] + inv[2]
    for k in range(3):
        w_ref[0, 0, k, :] = inv[k] / wsum


def _knn_topk(xyz_hi, xyz_lo_t, *, bn, interpret=False):
    B, N, _ = xyz_hi.shape
    S = xyz_lo_t.shape[2]
    grid = (B, N // bn)
    nb = N // bn
    out_shape = [
        jax.ShapeDtypeStruct((B, nb, 3, bn), jnp.int32),
        jax.ShapeDtypeStruct((B, nb, 3, bn), jnp.float32),
    ]
    return pl.pallas_call(
        functools.partial(_knn_body, S=S),
        grid=grid,
        in_specs=[
            pl.BlockSpec((1, bn, 3), lambda b, i: (b, i, 0)),
            pl.BlockSpec((1, 3, S), lambda b, i: (b, 0, 0)),
        ],
        out_specs=[
            pl.BlockSpec((1, 1, 3, bn), lambda b, i: (b, i, 0, 0)),
            pl.BlockSpec((1, 1, 3, bn), lambda b, i: (b, i, 0, 0)),
        ],
        out_shape=out_shape,
        interpret=interpret,
    )(xyz_hi, xyz_lo_t)


def _sc_gather(table, idx_flat, *, chunk):
    R = idx_flat.shape[0]
    D = table.shape[1]
    info = plsc.get_sparse_core_info()
    NC, NS = info.num_cores, info.num_subcores
    NW = NC * NS
    per_w = R // NW
    n_chunks = per_w // chunk
    mesh = plsc.VectorSubcoreMesh(core_axis_name="c", subcore_axis_name="s")

    @functools.partial(
        pl.kernel,
        out_type=jax.ShapeDtypeStruct((R, D), jnp.float32),
        mesh=mesh,
        scratch_types=[
            pltpu.VMEM((chunk,), jnp.int32),
            pltpu.VMEM((chunk, D), jnp.float32),
            pltpu.SemaphoreType.DMA,
        ],
    )
    def gather_kernel(idx_hbm, table_hbm, out_hbm, idx_v, rows_v, sem):
        wid = lax.axis_index("s") * NC + lax.axis_index("c")
        base = wid * per_w

        def body(j, carry):
            off = base + j * chunk
            pltpu.sync_copy(idx_hbm.at[pl.ds(off, chunk)], idx_v)
            pltpu.async_copy(table_hbm.at[idx_v], rows_v, sem).wait()
            pltpu.sync_copy(rows_v, out_hbm.at[pl.ds(off, chunk)])
            return carry

        lax.fori_loop(0, n_chunks, body, 0)

    return gather_kernel(idx_flat, table)


def _mlp_body(kf_ref, fs_ref, w_ref, W1a_ref, W1b_ref, b1_ref, W2_ref,
              b2_ref, g_ref, be_ref, o_ref):
    kf = kf_ref[...]
    w = w_ref[0]
    interp = (w[0][:, None] * kf[0] + w[1][:, None] * kf[1]
              + w[2][:, None] * kf[2])
    fs = fs_ref[...]
    h = (jnp.dot(fs, W1a_ref[...], preferred_element_type=jnp.float32)
         + jnp.dot(interp, W1b_ref[...], preferred_element_type=jnp.float32)
         + b1_ref[...])
    h = jnp.maximum(h, 0.0)
    h = jnp.dot(h, W2_ref[...], preferred_element_type=jnp.float32) + b2_ref[...]
    mu = jnp.mean(h, axis=-1, keepdims=True)
    hc = h - mu
    var = jnp.mean(hc * hc, axis=-1, keepdims=True)
    o_ref[...] = hc * lax.rsqrt(var + 1e-5) * g_ref[...] + be_ref[...]


def _mlp(knn_feat, feat_skip_f, w_t, W1a, W1b, b1, W2, b2, gamma, beta, *,
         bn2, interpret=False):
    BN, skip = feat_skip_f.shape
    D = knn_feat.shape[2]
    out_dim = W2.shape[1]
    nb = BN // bn2
    return pl.pallas_call(
        _mlp_body,
        grid=(nb,),
        in_specs=[
            pl.BlockSpec((3, bn2, D), lambda i: (0, i, 0)),
            pl.BlockSpec((bn2, skip), lambda i: (i, 0)),
            pl.BlockSpec((1, 3, bn2), lambda i: (i, 0, 0)),
            pl.BlockSpec((skip, out_dim), lambda i: (0, 0)),
            pl.BlockSpec((D, out_dim), lambda i: (0, 0)),
            pl.BlockSpec((1, out_dim), lambda i: (0, 0)),
            pl.BlockSpec((out_dim, out_dim), lambda i: (0, 0)),
            pl.BlockSpec((1, out_dim), lambda i: (0, 0)),
            pl.BlockSpec((1, out_dim), lambda i: (0, 0)),
            pl.BlockSpec((1, out_dim), lambda i: (0, 0)),
        ],
        out_specs=pl.BlockSpec((bn2, out_dim), lambda i: (i, 0)),
        out_shape=jax.ShapeDtypeStruct((BN, out_dim), jnp.float32),
        interpret=interpret,
    )(knn_feat, feat_skip_f, w_t, W1a, W1b, b1, W2, b2, gamma, beta)


def kernel(xyz_hi, xyz_lo, feat_skip, feat_lo, W1, b1, W2, b2, gamma, beta):
    B, N, _ = xyz_hi.shape
    S = xyz_lo.shape[1]
    low_dim = feat_lo.shape[2]
    skip_dim = feat_skip.shape[2]
    out_dim = W2.shape[1]
    bn, bn2 = 512, 512

    xyz_lo_t = jnp.swapaxes(xyz_lo, 1, 2)
    idx, w = _knn_topk(xyz_hi, xyz_lo_t, bn=bn)
    idx = idx.transpose(2, 0, 1, 3).reshape(3, B * N)
    w = w.transpose(2, 0, 1, 3).reshape(3, B * N)

    table = feat_lo.reshape(B * S, low_dim)
    idx_flat = idx.reshape(3 * B * N)
    knn_feat = _sc_gather(table, idx_flat, chunk=128)
    knn_feat = knn_feat.reshape(3, B * N, low_dim)

    nb2 = (B * N) // bn2
    w_t = w.reshape(3, nb2, bn2).transpose(1, 0, 2)

    out = _mlp(
        knn_feat,
        feat_skip.reshape(B * N, skip_dim),
        w_t,
        W1[:skip_dim], W1[skip_dim:],
        b1.reshape(1, out_dim),
        W2,
        b2.reshape(1, out_dim),
        gamma.reshape(1, out_dim),
        beta.reshape(1, out_dim),
        bn2=bn2,
    )
    return out.reshape(B, N, out_dim)


# speedup vs baseline: 31.0264x; 1.1925x over previous
import functools

import jax
import jax.numpy as jnp
from jax import lax
from jax.experimental import pallas as pl
from jax.experimental.pallas import tpu as pltpu
from jax.experimental.pallas import tpu_sc as plsc


def _knn_body(hi_ref, lo_ref, idx_ref, w_ref, *, S):
    b = pl.program_id(0)
    hiT = hi_ref[0]
    lo = lo_ref[0]
    bn = hiT.shape[1]

    a2 = (hiT[0:1] * hiT[0:1] + hiT[1:2] * hiT[1:2]
          + hiT[2:3] * hiT[2:3])
    b2 = (lo[:, 0:1] * lo[:, 0:1] + lo[:, 1:2] * lo[:, 1:2]
          + lo[:, 2:3] * lo[:, 2:3])
    ab = jnp.dot(lo.astype(jnp.bfloat16), hiT.astype(jnp.bfloat16),
                 preferred_element_type=jnp.float32)
    d2 = jnp.maximum(a2 + b2 - 2.0 * ab, 0.0)

    iota = lax.broadcasted_iota(jnp.int32, (S, bn), 0)
    BIG = jnp.float32(3.0e38)
    dists = []
    for k in range(3):
        m = jnp.min(d2, axis=0, keepdims=True)
        first = jnp.min(jnp.where(d2 <= m, iota, S), axis=0,
                        keepdims=True)
        if k < 2:
            d2 = jnp.where(iota == first, BIG, d2)
        idx_ref[0, 0, k, :] = first[0] + b * S
        dists.append(jnp.sqrt(jnp.maximum(m[0], 0.0)))
    inv = [1.0 / (d + 1e-8) for d in dists]
    wsum = inv[0] + inv[1] + inv[2]
    for k in range(3):
        w_ref[0, 0, k, :] = inv[k] / wsum


def _knn_topk(xyz_hi_t, xyz_lo, *, bn, interpret=False):
    B, _, N = xyz_hi_t.shape
    S = xyz_lo.shape[1]
    grid = (B, N // bn)
    nb = N // bn
    out_shape = [
        jax.ShapeDtypeStruct((B, nb, 3, bn), jnp.int32),
        jax.ShapeDtypeStruct((B, nb, 3, bn), jnp.float32),
    ]
    return pl.pallas_call(
        functools.partial(_knn_body, S=S),
        grid=grid,
        in_specs=[
            pl.BlockSpec((1, 3, bn), lambda b, i: (b, 0, i)),
            pl.BlockSpec((1, S, 3), lambda b, i: (b, 0, 0)),
        ],
        out_specs=[
            pl.BlockSpec((1, 1, 3, bn), lambda b, i: (b, i, 0, 0)),
            pl.BlockSpec((1, 1, 3, bn), lambda b, i: (b, i, 0, 0)),
        ],
        out_shape=out_shape,
        interpret=interpret,
    )(xyz_hi_t, xyz_lo)


def _sc_gather(table, idx_flat, *, chunk):
    R = idx_flat.shape[0]
    D = table.shape[1]
    info = plsc.get_sparse_core_info()
    NC, NS = info.num_cores, info.num_subcores
    NW = NC * NS
    per_w = R // NW
    n_chunks = per_w // chunk
    mesh = plsc.VectorSubcoreMesh(core_axis_name="c", subcore_axis_name="s")

    @functools.partial(
        pl.kernel,
        out_type=jax.ShapeDtypeStruct((R, D), jnp.float32),
        mesh=mesh,
        scratch_types=[
            pltpu.VMEM((chunk,), jnp.int32),
            pltpu.VMEM((chunk, D), jnp.float32),
            pltpu.SemaphoreType.DMA,
        ],
    )
    def gather_kernel(idx_hbm, table_hbm, out_hbm, idx_v, rows_v, sem):
        wid = lax.axis_index("s") * NC + lax.axis_index("c")
        base = wid * per_w

        def body(j, carry):
            off = base + j * chunk
            pltpu.sync_copy(idx_hbm.at[pl.ds(off, chunk)], idx_v)
            pltpu.async_copy(table_hbm.at[idx_v], rows_v, sem).wait()
            pltpu.sync_copy(rows_v, out_hbm.at[pl.ds(off, chunk)])
            return carry

        lax.fori_loop(0, n_chunks, body, 0)

    return gather_kernel(idx_flat, table)


def _mlp_body(kf_ref, fs_ref, w_ref, W1a_ref, W1b_ref, b1_ref, W2_ref,
              b2_ref, g_ref, be_ref, o_ref):
    kf = kf_ref[...]
    w = w_ref[0]
    interp = (w[0][:, None] * kf[0] + w[1][:, None] * kf[1]
              + w[2][:, None] * kf[2])
    fs = fs_ref[...]
    h = (jnp.dot(fs, W1a_ref[...], preferred_element_type=jnp.float32)
         + jnp.dot(interp, W1b_ref[...], preferred_element_type=jnp.float32)
         + b1_ref[...])
    h = jnp.maximum(h, 0.0)
    h = jnp.dot(h, W2_ref[...], preferred_element_type=jnp.float32) + b2_ref[...]
    mu = jnp.mean(h, axis=-1, keepdims=True)
    hc = h - mu
    var = jnp.mean(hc * hc, axis=-1, keepdims=True)
    o_ref[...] = hc * lax.rsqrt(var + 1e-5) * g_ref[...] + be_ref[...]


def _mlp(knn_feat, feat_skip_f, w_t, W1a, W1b, b1, W2, b2, gamma, beta, *,
         bn2, interpret=False):
    BN, skip = feat_skip_f.shape
    D = knn_feat.shape[2]
    out_dim = W2.shape[1]
    nb = BN // bn2
    return pl.pallas_call(
        _mlp_body,
        grid=(nb,),
        in_specs=[
            pl.BlockSpec((3, bn2, D), lambda i: (0, i, 0)),
            pl.BlockSpec((bn2, skip), lambda i: (i, 0)),
            pl.BlockSpec((1, 3, bn2), lambda i: (i, 0, 0)),
            pl.BlockSpec((skip, out_dim), lambda i: (0, 0)),
            pl.BlockSpec((D, out_dim), lambda i: (0, 0)),
            pl.BlockSpec((1, out_dim), lambda i: (0, 0)),
            pl.BlockSpec((out_dim, out_dim), lambda i: (0, 0)),
            pl.BlockSpec((1, out_dim), lambda i: (0, 0)),
            pl.BlockSpec((1, out_dim), lambda i: (0, 0)),
            pl.BlockSpec((1, out_dim), lambda i: (0, 0)),
        ],
        out_specs=pl.BlockSpec((bn2, out_dim), lambda i: (i, 0)),
        out_shape=jax.ShapeDtypeStruct((BN, out_dim), jnp.float32),
        interpret=interpret,
    )(knn_feat, feat_skip_f, w_t, W1a, W1b, b1, W2, b2, gamma, beta)


def kernel(xyz_hi, xyz_lo, feat_skip, feat_lo, W1, b1, W2, b2, gamma, beta):
    B, N, _ = xyz_hi.shape
    S = xyz_lo.shape[1]
    low_dim = feat_lo.shape[2]
    skip_dim = feat_skip.shape[2]
    out_dim = W2.shape[1]
    bn, bn2 = 512, 512

    xyz_hi_t = jnp.swapaxes(xyz_hi, 1, 2)
    idx, w = _knn_topk(xyz_hi_t, xyz_lo, bn=bn)
    idx = idx.transpose(2, 0, 1, 3).reshape(3, B * N)
    w = w.transpose(2, 0, 1, 3).reshape(3, B * N)

    table = feat_lo.reshape(B * S, low_dim)
    idx_flat = idx.reshape(3 * B * N)
    knn_feat = _sc_gather(table, idx_flat, chunk=128)
    knn_feat = knn_feat.reshape(3, B * N, low_dim)

    nb2 = (B * N) // bn2
    w_t = w.reshape(3, nb2, bn2).transpose(1, 0, 2)

    out = _mlp(
        knn_feat,
        feat_skip.reshape(B * N, skip_dim),
        w_t,
        W1[:skip_dim], W1[skip_dim:],
        b1.reshape(1, out_dim),
        W2,
        b2.reshape(1, out_dim),
        gamma.reshape(1, out_dim),
        beta.reshape(1, out_dim),
        bn2=bn2,
    )
    return out.reshape(B, N, out_dim)


# speedup vs baseline: 32.5589x; 1.0494x over previous
import functools

import jax
import jax.numpy as jnp
from jax import lax
from jax.experimental import pallas as pl
from jax.experimental.pallas import tpu as pltpu
from jax.experimental.pallas import tpu_sc as plsc


def _knn_body(hi_ref, lo_ref, idx_ref, w_ref, *, S):
    b = pl.program_id(0)
    hiT = hi_ref[0]
    lo = lo_ref[0]
    bn = hiT.shape[1]

    a2 = (hiT[0:1] * hiT[0:1] + hiT[1:2] * hiT[1:2]
          + hiT[2:3] * hiT[2:3])
    b2 = (lo[:, 0:1] * lo[:, 0:1] + lo[:, 1:2] * lo[:, 1:2]
          + lo[:, 2:3] * lo[:, 2:3])
    ab = jnp.dot(lo.astype(jnp.bfloat16), hiT.astype(jnp.bfloat16),
                 preferred_element_type=jnp.float32)
    d2 = jnp.maximum(a2 + b2 - 2.0 * ab, 0.0)

    iota = lax.broadcasted_iota(jnp.int32, (S, bn), 0)
    BIG = jnp.float32(3.0e38)
    dists = []
    for k in range(3):
        m = jnp.min(d2, axis=0, keepdims=True)
        first = jnp.min(jnp.where(d2 <= m, iota, S), axis=0,
                        keepdims=True)
        if k < 2:
            d2 = jnp.where(iota == first, BIG, d2)
        idx_ref[0, 0, k, :] = first[0] + b * S
        dists.append(jnp.sqrt(jnp.maximum(m[0], 0.0)))
    inv = [1.0 / (d + 1e-8) for d in dists]
    wsum = inv[0] + inv[1] + inv[2]
    for k in range(3):
        w_ref[0, 0, k, :] = inv[k] / wsum


def _knn_topk(xyz_hi_t, xyz_lo, *, bn, interpret=False):
    B, _, N = xyz_hi_t.shape
    S = xyz_lo.shape[1]
    grid = (B, N // bn)
    nb = N // bn
    out_shape = [
        jax.ShapeDtypeStruct((B, nb, 3, bn), jnp.int32),
        jax.ShapeDtypeStruct((B, nb, 3, bn), jnp.float32),
    ]
    return pl.pallas_call(
        functools.partial(_knn_body, S=S),
        grid=grid,
        in_specs=[
            pl.BlockSpec((1, 3, bn), lambda b, i: (b, 0, i)),
            pl.BlockSpec((1, S, 3), lambda b, i: (b, 0, 0)),
        ],
        out_specs=[
            pl.BlockSpec((1, 1, 3, bn), lambda b, i: (b, i, 0, 0)),
            pl.BlockSpec((1, 1, 3, bn), lambda b, i: (b, i, 0, 0)),
        ],
        out_shape=out_shape,
        interpret=interpret,
    )(xyz_hi_t, xyz_lo)


def _sc_gather(table, idx3, *, nbuf=3):
    NW, n_chunks, chunk = idx3.shape
    D = table.shape[1]
    R = NW * n_chunks * chunk
    info = plsc.get_sparse_core_info()
    NC = info.num_cores
    mesh = plsc.VectorSubcoreMesh(core_axis_name="c", subcore_axis_name="s")

    @functools.partial(
        pl.kernel,
        out_type=jax.ShapeDtypeStruct((R, D), jnp.float32),
        mesh=mesh,
        scratch_types=[
            pltpu.VMEM((n_chunks, chunk), jnp.int32),
            pltpu.VMEM((nbuf, chunk, D), jnp.float32),
            pltpu.SemaphoreType.DMA((nbuf,)),
            pltpu.SemaphoreType.DMA((nbuf,)),
        ],
    )
    def gather_kernel(idx_hbm, table_hbm, out_hbm, idx_v, rows_v, g_sem, o_sem):
        wid = lax.axis_index("s") * NC + lax.axis_index("c")
        base = wid * (n_chunks * chunk)
        pltpu.sync_copy(idx_hbm.at[wid], idx_v)

        def gather(t):
            return pltpu.async_copy(
                table_hbm.at[idx_v.at[t]], rows_v.at[t % nbuf],
                g_sem.at[t % nbuf])

        def writeout(t):
            return pltpu.async_copy(
                rows_v.at[t % nbuf],
                out_hbm.at[pl.ds(base + t * chunk, chunk)],
                o_sem.at[t % nbuf])

        gcop, ocop = {}, {}
        for t in range(min(2, n_chunks)):
            gcop[t] = gather(t)
        for t in range(n_chunks):
            gcop[t].wait()
            ocop[t] = writeout(t)
            if t + 2 < n_chunks:
                if t >= 1:
                    ocop[t - 1].wait()
                gcop[t + 2] = gather(t + 2)
        for t in range(max(0, n_chunks - 2), n_chunks):
            ocop[t].wait()

    return gather_kernel(idx3, table)


def _mlp_body(kf_ref, fs_ref, w_ref, W1a_ref, W1b_ref, b1_ref, W2_ref,
              b2_ref, g_ref, be_ref, o_ref):
    kf = kf_ref[...]
    w = w_ref[0]
    interp = (w[0][:, None] * kf[0] + w[1][:, None] * kf[1]
              + w[2][:, None] * kf[2])
    fs = fs_ref[...]
    h = (jnp.dot(fs, W1a_ref[...], preferred_element_type=jnp.float32)
         + jnp.dot(interp, W1b_ref[...], preferred_element_type=jnp.float32)
         + b1_ref[...])
    h = jnp.maximum(h, 0.0)
    h = jnp.dot(h, W2_ref[...], preferred_element_type=jnp.float32) + b2_ref[...]
    mu = jnp.mean(h, axis=-1, keepdims=True)
    hc = h - mu
    var = jnp.mean(hc * hc, axis=-1, keepdims=True)
    o_ref[...] = hc * lax.rsqrt(var + 1e-5) * g_ref[...] + be_ref[...]


def _mlp(knn_feat, feat_skip_f, w_t, W1a, W1b, b1, W2, b2, gamma, beta, *,
         bn2, interpret=False):
    BN, skip = feat_skip_f.shape
    D = knn_feat.shape[2]
    out_dim = W2.shape[1]
    nb = BN // bn2
    return pl.pallas_call(
        _mlp_body,
        grid=(nb,),
        in_specs=[
            pl.BlockSpec((3, bn2, D), lambda i: (0, i, 0)),
            pl.BlockSpec((bn2, skip), lambda i: (i, 0)),
            pl.BlockSpec((1, 3, bn2), lambda i: (i, 0, 0)),
            pl.BlockSpec((skip, out_dim), lambda i: (0, 0)),
            pl.BlockSpec((D, out_dim), lambda i: (0, 0)),
            pl.BlockSpec((1, out_dim), lambda i: (0, 0)),
            pl.BlockSpec((out_dim, out_dim), lambda i: (0, 0)),
            pl.BlockSpec((1, out_dim), lambda i: (0, 0)),
            pl.BlockSpec((1, out_dim), lambda i: (0, 0)),
            pl.BlockSpec((1, out_dim), lambda i: (0, 0)),
        ],
        out_specs=pl.BlockSpec((bn2, out_dim), lambda i: (i, 0)),
        out_shape=jax.ShapeDtypeStruct((BN, out_dim), jnp.float32),
        interpret=interpret,
    )(knn_feat, feat_skip_f, w_t, W1a, W1b, b1, W2, b2, gamma, beta)


def kernel(xyz_hi, xyz_lo, feat_skip, feat_lo, W1, b1, W2, b2, gamma, beta):
    B, N, _ = xyz_hi.shape
    S = xyz_lo.shape[1]
    low_dim = feat_lo.shape[2]
    skip_dim = feat_skip.shape[2]
    out_dim = W2.shape[1]
    bn, bn2 = 512, 512

    xyz_hi_t = jnp.swapaxes(xyz_hi, 1, 2)
    idx, w = _knn_topk(xyz_hi_t, xyz_lo, bn=bn)
    idx = idx.transpose(2, 0, 1, 3).reshape(3, B * N)
    w = w.transpose(2, 0, 1, 3).reshape(3, B * N)

    table = feat_lo.reshape(B * S, low_dim)
    NW, chunk = 32, 128
    idx3 = idx.reshape(NW, (3 * B * N) // (NW * chunk), chunk)
    knn_feat = _sc_gather(table, idx3)
    knn_feat = knn_feat.reshape(3, B * N, low_dim)

    nb2 = (B * N) // bn2
    w_t = w.reshape(3, nb2, bn2).transpose(1, 0, 2)

    out = _mlp(
        knn_feat,
        feat_skip.reshape(B * N, skip_dim),
        w_t,
        W1[:skip_dim], W1[skip_dim:],
        b1.reshape(1, out_dim),
        W2,
        b2.reshape(1, out_dim),
        gamma.reshape(1, out_dim),
        beta.reshape(1, out_dim),
        bn2=bn2,
    )
    return out.reshape(B, N, out_dim)


# speedup vs baseline: 32.5975x; 1.0012x over previous
import functools

import jax
import jax.numpy as jnp
from jax import lax
from jax.experimental import pallas as pl
from jax.experimental.pallas import tpu as pltpu
from jax.experimental.pallas import tpu_sc as plsc


def _knn_body(hi_ref, lo_ref, idx_ref, w_ref, *, S):
    b = pl.program_id(0)
    hiT = hi_ref[0]
    lo = lo_ref[0]
    bn = hiT.shape[1]

    a2 = (hiT[0:1] * hiT[0:1] + hiT[1:2] * hiT[1:2]
          + hiT[2:3] * hiT[2:3])
    b2 = (lo[:, 0:1] * lo[:, 0:1] + lo[:, 1:2] * lo[:, 1:2]
          + lo[:, 2:3] * lo[:, 2:3])
    ab = jnp.dot(lo.astype(jnp.bfloat16), hiT.astype(jnp.bfloat16),
                 preferred_element_type=jnp.float32)
    d2 = jnp.maximum(a2 + b2 - 2.0 * ab, 0.0)

    iota = lax.broadcasted_iota(jnp.int32, (S, bn), 0)
    BIG = jnp.float32(3.0e38)
    dists = []
    for k in range(3):
        m = jnp.min(d2, axis=0, keepdims=True)
        first = jnp.min(jnp.where(d2 <= m, iota, S), axis=0,
                        keepdims=True)
        if k < 2:
            d2 = jnp.where(iota == first, BIG, d2)
        idx_ref[0, 0, k, :] = first[0] + b * S
        dists.append(jnp.sqrt(jnp.maximum(m[0], 0.0)))
    inv = [1.0 / (d + 1e-8) for d in dists]
    wsum = inv[0] + inv[1] + inv[2]
    for k in range(3):
        w_ref[0, 0, k, :] = inv[k] / wsum


def _knn_topk(xyz_hi_t, xyz_lo, *, bn, interpret=False):
    B, _, N = xyz_hi_t.shape
    S = xyz_lo.shape[1]
    grid = (B, N // bn)
    nb = N // bn
    out_shape = [
        jax.ShapeDtypeStruct((B, nb, 3, bn), jnp.int32),
        jax.ShapeDtypeStruct((B, nb, 3, bn), jnp.float32),
    ]
    return pl.pallas_call(
        functools.partial(_knn_body, S=S),
        grid=grid,
        in_specs=[
            pl.BlockSpec((1, 3, bn), lambda b, i: (b, 0, i)),
            pl.BlockSpec((1, S, 3), lambda b, i: (b, 0, 0)),
        ],
        out_specs=[
            pl.BlockSpec((1, 1, 3, bn), lambda b, i: (b, i, 0, 0)),
            pl.BlockSpec((1, 1, 3, bn), lambda b, i: (b, i, 0, 0)),
        ],
        out_shape=out_shape,
        interpret=interpret,
    )(xyz_hi_t, xyz_lo)


def _sc_gather(table, idx3, *, nbuf=3):
    NW, n_chunks, chunk = idx3.shape
    D = table.shape[1]
    R = NW * n_chunks * chunk
    info = plsc.get_sparse_core_info()
    NC = info.num_cores
    mesh = plsc.VectorSubcoreMesh(core_axis_name="c", subcore_axis_name="s")

    @functools.partial(
        pl.kernel,
        out_type=jax.ShapeDtypeStruct((R, D), jnp.float32),
        mesh=mesh,
        scratch_types=[
            pltpu.VMEM((n_chunks, chunk), jnp.int32),
            pltpu.VMEM((nbuf, chunk, D), jnp.float32),
            pltpu.SemaphoreType.DMA((nbuf,)),
            pltpu.SemaphoreType.DMA((nbuf,)),
        ],
    )
    def gather_kernel(idx_hbm, table_hbm, out_hbm, idx_v, rows_v, g_sem, o_sem):
        wid = lax.axis_index("s") * NC + lax.axis_index("c")
        base = wid * (n_chunks * chunk)
        pltpu.sync_copy(idx_hbm.at[wid], idx_v)

        def gather(t):
            return pltpu.async_copy(
                table_hbm.at[idx_v.at[t]], rows_v.at[t % nbuf],
                g_sem.at[t % nbuf])

        def writeout(t):
            return pltpu.async_copy(
                rows_v.at[t % nbuf],
                out_hbm.at[pl.ds(base + t * chunk, chunk)],
                o_sem.at[t % nbuf])

        gcop, ocop = {}, {}
        for t in range(min(2, n_chunks)):
            gcop[t] = gather(t)
        for t in range(n_chunks):
            gcop[t].wait()
            ocop[t] = writeout(t)
            if t + 2 < n_chunks:
                if t >= 1:
                    ocop[t - 1].wait()
                gcop[t + 2] = gather(t + 2)
        for t in range(max(0, n_chunks - 2), n_chunks):
            ocop[t].wait()

    return gather_kernel(idx3, table)


def _mlp_body(kf_ref, fs_ref, w_ref, W1a_ref, W1b_ref, b1_ref, W2_ref,
              b2_ref, g_ref, be_ref, o_ref):
    kf = kf_ref[0, 0]
    w = w_ref[0, 0]
    interp = (w[0][:, None] * kf[0] + w[1][:, None] * kf[1]
              + w[2][:, None] * kf[2])
    fs = fs_ref[0]
    h = (jnp.dot(fs, W1a_ref[...], preferred_element_type=jnp.float32)
         + jnp.dot(interp, W1b_ref[...], preferred_element_type=jnp.float32)
         + b1_ref[...])
    h = jnp.maximum(h, 0.0)
    h = jnp.dot(h, W2_ref[...], preferred_element_type=jnp.float32) + b2_ref[...]
    mu = jnp.mean(h, axis=-1, keepdims=True)
    hc = h - mu
    var = jnp.mean(hc * hc, axis=-1, keepdims=True)
    o_ref[0] = hc * lax.rsqrt(var + 1e-5) * g_ref[...] + be_ref[...]


def _mlp(knn_feat5, feat_skip, w4, W1a, W1b, b1, W2, b2, gamma, beta, *,
         interpret=False):
    B, nb, _, bn, D = knn_feat5.shape
    skip = feat_skip.shape[2]
    out_dim = W2.shape[1]
    N = nb * bn
    return pl.pallas_call(
        _mlp_body,
        grid=(B, nb),
        in_specs=[
            pl.BlockSpec((1, 1, 3, bn, D), lambda b, i: (b, i, 0, 0, 0)),
            pl.BlockSpec((1, bn, skip), lambda b, i: (b, i, 0)),
            pl.BlockSpec((1, 1, 3, bn), lambda b, i: (b, i, 0, 0)),
            pl.BlockSpec((skip, out_dim), lambda b, i: (0, 0)),
            pl.BlockSpec((D, out_dim), lambda b, i: (0, 0)),
            pl.BlockSpec((1, out_dim), lambda b, i: (0, 0)),
            pl.BlockSpec((out_dim, out_dim), lambda b, i: (0, 0)),
            pl.BlockSpec((1, out_dim), lambda b, i: (0, 0)),
            pl.BlockSpec((1, out_dim), lambda b, i: (0, 0)),
            pl.BlockSpec((1, out_dim), lambda b, i: (0, 0)),
        ],
        out_specs=pl.BlockSpec((1, bn, out_dim), lambda b, i: (b, i, 0)),
        out_shape=jax.ShapeDtypeStruct((B, N, out_dim), jnp.float32),
        interpret=interpret,
    )(knn_feat5, feat_skip, w4, W1a, W1b, b1, W2, b2, gamma, beta)


def kernel(xyz_hi, xyz_lo, feat_skip, feat_lo, W1, b1, W2, b2, gamma, beta):
    B, N, _ = xyz_hi.shape
    S = xyz_lo.shape[1]
    low_dim = feat_lo.shape[2]
    skip_dim = feat_skip.shape[2]
    out_dim = W2.shape[1]
    bn, bn2 = 512, 512

    xyz_hi_t = jnp.swapaxes(xyz_hi, 1, 2)
    idx, w = _knn_topk(xyz_hi_t, xyz_lo, bn=bn)

    table = feat_lo.reshape(B * S, low_dim)
    NW, chunk = 32, 128
    nb = N // bn
    idx3 = idx.reshape(NW, (3 * B * N) // (NW * chunk), chunk)
    knn_feat5 = _sc_gather(table, idx3).reshape(B, nb, 3, bn, low_dim)

    return _mlp(
        knn_feat5,
        feat_skip,
        w,
        W1[:skip_dim], W1[skip_dim:],
        b1.reshape(1, out_dim),
        W2,
        b2.reshape(1, out_dim),
        gamma.reshape(1, out_dim),
        beta.reshape(1, out_dim),
    )


# speedup vs baseline: 33.1025x; 1.0155x over previous
import functools

import jax
import jax.numpy as jnp
from jax import lax
from jax.experimental import pallas as pl
from jax.experimental.pallas import tpu as pltpu
from jax.experimental.pallas import tpu_sc as plsc


def _knn_body(hi_ref, lo_ref, idx_ref, w_ref, *, S):
    b = pl.program_id(0)
    hiT = hi_ref[0]
    lo = lo_ref[0]
    bn = hiT.shape[1]

    a2 = (hiT[0:1] * hiT[0:1] + hiT[1:2] * hiT[1:2]
          + hiT[2:3] * hiT[2:3])
    b2 = (lo[:, 0:1] * lo[:, 0:1] + lo[:, 1:2] * lo[:, 1:2]
          + lo[:, 2:3] * lo[:, 2:3])
    ab = jnp.dot(lo.astype(jnp.bfloat16), hiT.astype(jnp.bfloat16),
                 preferred_element_type=jnp.float32)
    d2 = jnp.maximum(a2 + b2 - 2.0 * ab, 0.0)

    iota = lax.broadcasted_iota(jnp.int32, (S, bn), 0)
    BIG = jnp.float32(3.0e38)
    dists = []
    for k in range(3):
        m = jnp.min(d2, axis=0, keepdims=True)
        first = jnp.min(jnp.where(d2 <= m, iota, S), axis=0,
                        keepdims=True)
        if k < 2:
            d2 = jnp.where(iota == first, BIG, d2)
        idx_ref[0, 0, k, :] = first[0] + b * S
        dists.append(jnp.sqrt(jnp.maximum(m[0], 0.0)))
    inv = [1.0 / (d + 1e-8) for d in dists]
    wsum = inv[0] + inv[1] + inv[2]
    for k in range(3):
        w_ref[0, 0, k, :] = inv[k] / wsum


def _knn_topk(xyz_hi_t, xyz_lo, *, bn, interpret=False):
    B, _, N = xyz_hi_t.shape
    S = xyz_lo.shape[1]
    grid = (B, N // bn)
    nb = N // bn
    out_shape = [
        jax.ShapeDtypeStruct((B, nb, 3, bn), jnp.int32),
        jax.ShapeDtypeStruct((B, nb, 3, bn), jnp.float32),
    ]
    return pl.pallas_call(
        functools.partial(_knn_body, S=S),
        grid=grid,
        in_specs=[
            pl.BlockSpec((1, 3, bn), lambda b, i: (b, 0, i)),
            pl.BlockSpec((1, S, 3), lambda b, i: (b, 0, 0)),
        ],
        out_specs=[
            pl.BlockSpec((1, 1, 3, bn), lambda b, i: (b, i, 0, 0)),
            pl.BlockSpec((1, 1, 3, bn), lambda b, i: (b, i, 0, 0)),
        ],
        out_shape=out_shape,
        interpret=interpret,
    )(xyz_hi_t, xyz_lo)


def _sc_gather(table, idx3, *, nbuf=3):
    NW, n_chunks, chunk = idx3.shape
    D = table.shape[1]
    R = NW * n_chunks * chunk
    info = plsc.get_sparse_core_info()
    NC = info.num_cores
    mesh = plsc.VectorSubcoreMesh(core_axis_name="c", subcore_axis_name="s")

    @functools.partial(
        pl.kernel,
        out_type=jax.ShapeDtypeStruct((R, D), jnp.float32),
        mesh=mesh,
        scratch_types=[
            pltpu.VMEM((n_chunks, chunk), jnp.int32),
            pltpu.VMEM((nbuf, chunk, D), jnp.float32),
            pltpu.SemaphoreType.DMA((nbuf,)),
            pltpu.SemaphoreType.DMA((nbuf,)),
        ],
    )
    def gather_kernel(idx_hbm, table_hbm, out_hbm, idx_v, rows_v, g_sem, o_sem):
        wid = lax.axis_index("s") * NC + lax.axis_index("c")
        base = wid * (n_chunks * chunk)
        pltpu.sync_copy(idx_hbm.at[wid], idx_v)

        def gather(t):
            return pltpu.async_copy(
                table_hbm.at[idx_v.at[t]], rows_v.at[t % nbuf],
                g_sem.at[t % nbuf])

        def writeout(t):
            return pltpu.async_copy(
                rows_v.at[t % nbuf],
                out_hbm.at[pl.ds(base + t * chunk, chunk)],
                o_sem.at[t % nbuf])

        gcop, ocop = {}, {}
        for t in range(min(2, n_chunks)):
            gcop[t] = gather(t)
        for t in range(n_chunks):
            gcop[t].wait()
            ocop[t] = writeout(t)
            if t + 2 < n_chunks:
                if t >= 1:
                    ocop[t - 1].wait()
                gcop[t + 2] = gather(t + 2)
        for t in range(max(0, n_chunks - 2), n_chunks):
            ocop[t].wait()

    return gather_kernel(idx3, table)


def _mlp_body(kf_ref, fs_ref, w_ref, W1a_ref, W1b_ref, b1_ref, W2_ref,
              b2_ref, g_ref, be_ref, o_ref):
    kf = kf_ref[0, 0]
    w = w_ref[0, 0]
    interp = (w[0][:, None] * kf[0] + w[1][:, None] * kf[1]
              + w[2][:, None] * kf[2])
    fs = fs_ref[0]
    h = (jnp.dot(fs, W1a_ref[...], preferred_element_type=jnp.float32)
         + jnp.dot(interp, W1b_ref[...], preferred_element_type=jnp.float32)
         + b1_ref[...])
    h = jnp.maximum(h, 0.0)
    h = jnp.dot(h, W2_ref[...], preferred_element_type=jnp.float32) + b2_ref[...]
    mu = jnp.mean(h, axis=-1, keepdims=True)
    hc = h - mu
    var = jnp.mean(hc * hc, axis=-1, keepdims=True)
    o_ref[0] = hc * lax.rsqrt(var + 1e-5) * g_ref[...] + be_ref[...]


def _mlp(knn_feat5, feat_skip, w4, W1a, W1b, b1, W2, b2, gamma, beta, *,
         interpret=False):
    B, nb, _, bn, D = knn_feat5.shape
    skip = feat_skip.shape[2]
    out_dim = W2.shape[1]
    N = nb * bn
    return pl.pallas_call(
        _mlp_body,
        grid=(B, nb),
        in_specs=[
            pl.BlockSpec((1, 1, 3, bn, D), lambda b, i: (b, i, 0, 0, 0)),
            pl.BlockSpec((1, bn, skip), lambda b, i: (b, i, 0)),
            pl.BlockSpec((1, 1, 3, bn), lambda b, i: (b, i, 0, 0)),
            pl.BlockSpec((skip, out_dim), lambda b, i: (0, 0)),
            pl.BlockSpec((D, out_dim), lambda b, i: (0, 0)),
            pl.BlockSpec((1, out_dim), lambda b, i: (0, 0)),
            pl.BlockSpec((out_dim, out_dim), lambda b, i: (0, 0)),
            pl.BlockSpec((1, out_dim), lambda b, i: (0, 0)),
            pl.BlockSpec((1, out_dim), lambda b, i: (0, 0)),
            pl.BlockSpec((1, out_dim), lambda b, i: (0, 0)),
        ],
        out_specs=pl.BlockSpec((1, bn, out_dim), lambda b, i: (b, i, 0)),
        out_shape=jax.ShapeDtypeStruct((B, N, out_dim), jnp.float32),
        interpret=interpret,
    )(knn_feat5, feat_skip, w4, W1a, W1b, b1, W2, b2, gamma, beta)


def kernel(xyz_hi, xyz_lo, feat_skip, feat_lo, W1, b1, W2, b2, gamma, beta):
    B, N, _ = xyz_hi.shape
    S = xyz_lo.shape[1]
    low_dim = feat_lo.shape[2]
    skip_dim = feat_skip.shape[2]
    out_dim = W2.shape[1]
    bn, bn2 = 512, 512

    xyz_hi_t = jnp.swapaxes(xyz_hi, 1, 2)
    table = feat_lo.reshape(B * S, low_dim)
    NW, chunk = 32, 128

    h = N // 2
    nb = h // bn
    parts = []
    for p in range(2):
        sl = slice(p * h, (p + 1) * h)
        idx, w = _knn_topk(xyz_hi_t[:, :, sl], xyz_lo, bn=bn)
        idx3 = idx.reshape(NW, (3 * B * h) // (NW * chunk), chunk)
        knn_feat5 = _sc_gather(table, idx3).reshape(B, nb, 3, bn, low_dim)
        parts.append(_mlp(
            knn_feat5,
            feat_skip[:, sl],
            w,
            W1[:skip_dim], W1[skip_dim:],
            b1.reshape(1, out_dim),
            W2,
            b2.reshape(1, out_dim),
            gamma.reshape(1, out_dim),
            beta.reshape(1, out_dim),
        ))
    return jnp.concatenate(parts, axis=1)


# speedup vs baseline: 33.1207x; 1.0005x over previous
import functools

import jax
import jax.numpy as jnp
from jax import lax
from jax.experimental import pallas as pl
from jax.experimental.pallas import tpu as pltpu
from jax.experimental.pallas import tpu_sc as plsc


def _knn_body(hi_ref, lo_ref, idx_ref, w_ref, *, S):
    b = pl.program_id(0)
    hiT = hi_ref[0]
    lo = lo_ref[0]
    bn = hiT.shape[1]

    a2 = (hiT[0:1] * hiT[0:1] + hiT[1:2] * hiT[1:2]
          + hiT[2:3] * hiT[2:3])
    b2 = (lo[:, 0:1] * lo[:, 0:1] + lo[:, 1:2] * lo[:, 1:2]
          + lo[:, 2:3] * lo[:, 2:3])
    ab = jnp.dot(lo.astype(jnp.bfloat16), hiT.astype(jnp.bfloat16),
                 preferred_element_type=jnp.float32)
    d2 = jnp.maximum(a2 + b2 - 2.0 * ab, 0.0)

    iota = lax.broadcasted_iota(jnp.int32, (S, bn), 0)
    BIG = jnp.float32(3.0e38)
    dists = []
    for k in range(3):
        m = jnp.min(d2, axis=0, keepdims=True)
        first = jnp.min(jnp.where(d2 <= m, iota, S), axis=0,
                        keepdims=True)
        if k < 2:
            d2 = jnp.where(iota == first, BIG, d2)
        idx_ref[0, 0, k, :] = first[0] + b * S
        dists.append(jnp.sqrt(jnp.maximum(m[0], 0.0)))
    inv = [1.0 / (d + 1e-8) for d in dists]
    wsum = inv[0] + inv[1] + inv[2]
    for k in range(3):
        w_ref[0, 0, k, :] = inv[k] / wsum


def _knn_topk(xyz_hi_t, xyz_lo, *, bn, interpret=False):
    B, _, N = xyz_hi_t.shape
    S = xyz_lo.shape[1]
    grid = (B, N // bn)
    nb = N // bn
    out_shape = [
        jax.ShapeDtypeStruct((B, nb, 3, bn), jnp.int32),
        jax.ShapeDtypeStruct((B, nb, 3, bn), jnp.float32),
    ]
    return pl.pallas_call(
        functools.partial(_knn_body, S=S),
        grid=grid,
        in_specs=[
            pl.BlockSpec((1, 3, bn), lambda b, i: (b, 0, i)),
            pl.BlockSpec((1, S, 3), lambda b, i: (b, 0, 0)),
        ],
        out_specs=[
            pl.BlockSpec((1, 1, 3, bn), lambda b, i: (b, i, 0, 0)),
            pl.BlockSpec((1, 1, 3, bn), lambda b, i: (b, i, 0, 0)),
        ],
        out_shape=out_shape,
        interpret=interpret,
    )(xyz_hi_t, xyz_lo)


def _sc_gather(table, idx3, *, nbuf=3):
    NW, n_chunks, chunk = idx3.shape
    D = table.shape[1]
    R = NW * n_chunks * chunk
    info = plsc.get_sparse_core_info()
    NC = info.num_cores
    mesh = plsc.VectorSubcoreMesh(core_axis_name="c", subcore_axis_name="s")

    @functools.partial(
        pl.kernel,
        out_type=jax.ShapeDtypeStruct((R, D), jnp.float32),
        mesh=mesh,
        scratch_types=[
            pltpu.VMEM((n_chunks, chunk), jnp.int32),
            pltpu.VMEM((nbuf, chunk, D), jnp.float32),
            pltpu.SemaphoreType.DMA((nbuf,)),
            pltpu.SemaphoreType.DMA((nbuf,)),
        ],
    )
    def gather_kernel(idx_hbm, table_hbm, out_hbm, idx_v, rows_v, g_sem, o_sem):
        wid = lax.axis_index("s") * NC + lax.axis_index("c")
        base = wid * (n_chunks * chunk)
        pltpu.sync_copy(idx_hbm.at[wid], idx_v)

        def gather(t):
            return pltpu.async_copy(
                table_hbm.at[idx_v.at[t]], rows_v.at[t % nbuf],
                g_sem.at[t % nbuf])

        def writeout(t):
            return pltpu.async_copy(
                rows_v.at[t % nbuf],
                out_hbm.at[pl.ds(base + t * chunk, chunk)],
                o_sem.at[t % nbuf])

        gcop, ocop = {}, {}
        for t in range(min(2, n_chunks)):
            gcop[t] = gather(t)
        for t in range(n_chunks):
            gcop[t].wait()
            ocop[t] = writeout(t)
            if t + 2 < n_chunks:
                if t >= 1:
                    ocop[t - 1].wait()
                gcop[t + 2] = gather(t + 2)
        for t in range(max(0, n_chunks - 3), n_chunks):
            ocop[t].wait()

    return gather_kernel(idx3, table)


def _mlp_body(kf_ref, fs_ref, w_ref, W1a_ref, W1b_ref, b1_ref, W2_ref,
              b2_ref, g_ref, be_ref, o_ref):
    kf = kf_ref[0, 0]
    w = w_ref[0, 0]
    interp = (w[0][:, None] * kf[0] + w[1][:, None] * kf[1]
              + w[2][:, None] * kf[2])
    fs = fs_ref[0]
    h = (jnp.dot(fs, W1a_ref[...], preferred_element_type=jnp.float32)
         + jnp.dot(interp, W1b_ref[...], preferred_element_type=jnp.float32)
         + b1_ref[...])
    h = jnp.maximum(h, 0.0)
    h = jnp.dot(h, W2_ref[...], preferred_element_type=jnp.float32) + b2_ref[...]
    mu = jnp.mean(h, axis=-1, keepdims=True)
    hc = h - mu
    var = jnp.mean(hc * hc, axis=-1, keepdims=True)
    o_ref[0] = hc * lax.rsqrt(var + 1e-5) * g_ref[...] + be_ref[...]


def _mlp(knn_feat5, feat_skip, w4, W1a, W1b, b1, W2, b2, gamma, beta, *,
         interpret=False):
    B, nb, _, bn, D = knn_feat5.shape
    skip = feat_skip.shape[2]
    out_dim = W2.shape[1]
    N = nb * bn
    return pl.pallas_call(
        _mlp_body,
        grid=(B, nb),
        in_specs=[
            pl.BlockSpec((1, 1, 3, bn, D), lambda b, i: (b, i, 0, 0, 0)),
            pl.BlockSpec((1, bn, skip), lambda b, i: (b, i, 0)),
            pl.BlockSpec((1, 1, 3, bn), lambda b, i: (b, i, 0, 0)),
            pl.BlockSpec((skip, out_dim), lambda b, i: (0, 0)),
            pl.BlockSpec((D, out_dim), lambda b, i: (0, 0)),
            pl.BlockSpec((1, out_dim), lambda b, i: (0, 0)),
            pl.BlockSpec((out_dim, out_dim), lambda b, i: (0, 0)),
            pl.BlockSpec((1, out_dim), lambda b, i: (0, 0)),
            pl.BlockSpec((1, out_dim), lambda b, i: (0, 0)),
            pl.BlockSpec((1, out_dim), lambda b, i: (0, 0)),
        ],
        out_specs=pl.BlockSpec((1, bn, out_dim), lambda b, i: (b, i, 0)),
        out_shape=jax.ShapeDtypeStruct((B, N, out_dim), jnp.float32),
        interpret=interpret,
    )(knn_feat5, feat_skip, w4, W1a, W1b, b1, W2, b2, gamma, beta)


def kernel(xyz_hi, xyz_lo, feat_skip, feat_lo, W1, b1, W2, b2, gamma, beta):
    B, N, _ = xyz_hi.shape
    S = xyz_lo.shape[1]
    low_dim = feat_lo.shape[2]
    skip_dim = feat_skip.shape[2]
    out_dim = W2.shape[1]
    bn, bn2 = 512, 512

    xyz_hi_t = jnp.swapaxes(xyz_hi, 1, 2)
    table = feat_lo.reshape(B * S, low_dim)
    NW, chunk = 32, 128

    h = N // 2
    nb = h // bn
    parts = []
    for p in range(2):
        sl = slice(p * h, (p + 1) * h)
        idx, w = _knn_topk(xyz_hi_t[:, :, sl], xyz_lo, bn=bn)
        idx3 = idx.reshape(NW, (3 * B * h) // (NW * chunk), chunk)
        knn_feat5 = _sc_gather(table, idx3).reshape(B, nb, 3, bn, low_dim)
        parts.append(_mlp(
            knn_feat5,
            feat_skip[:, sl],
            w,
            W1[:skip_dim], W1[skip_dim:],
            b1.reshape(1, out_dim),
            W2,
            b2.reshape(1, out_dim),
            gamma.reshape(1, out_dim),
            beta.reshape(1, out_dim),
        ))
    return jnp.concatenate(parts, axis=1)
